# baseline (speedup 1.0000x reference)
import functools

import jax
import jax.numpy as jnp
from jax import lax
from jax.experimental import pallas as pl
from jax.experimental.pallas import tpu as pltpu
from jax.experimental.pallas import tpu_sc as plsc

S = 16384
HL = 1024
HR = 256
W = HL + 2 * HR

NC = 2
NS = 16
NW = NC * NS
BW = S // NW
R = 32
NCH = BW // R


@functools.partial(
    pl.kernel,
    mesh=plsc.VectorSubcoreMesh(core_axis_name="c", subcore_axis_name="s"),
    out_type=jax.ShapeDtypeStruct((S, W), jnp.float32),
    scratch_types=[
        pltpu.VMEM((NCH, R), jnp.int32),
        pltpu.VMEM((NCH, R), jnp.int32),
        pltpu.VMEM((NCH, R), jnp.int32),
        pltpu.VMEM((R, W), jnp.float32),
        pltpu.VMEM((R, W), jnp.float32),
        pltpu.SemaphoreType.DMA,
        pltpu.SemaphoreType.DMA,
        pltpu.SemaphoreType.DMA,
        pltpu.SemaphoreType.DMA,
    ],
)
def _sc_embed(lab_i_hbm, row_i_hbm, col_i_hbm, lab_w_hbm, row_w_hbm,
              col_w_hbm, out_hbm, lab_i, row_i, col_i, buf0, buf1,
              gsem0, gsem1, ssem0, ssem1):
    wid = lax.axis_index("s") * NC + lax.axis_index("c")
    pltpu.sync_copy(lab_i_hbm.at[wid], lab_i)
    pltpu.sync_copy(row_i_hbm.at[wid], row_i)
    pltpu.sync_copy(col_i_hbm.at[wid], col_i)

    def gather(c, buf, sem):
        c1 = pltpu.async_copy(
            lab_w_hbm.at[lab_i.at[c]], buf.at[:, pl.ds(0, HL)], sem)
        c2 = pltpu.async_copy(
            row_w_hbm.at[row_i.at[c]], buf.at[:, pl.ds(HL, HR)], sem)
        c3 = pltpu.async_copy(
            col_w_hbm.at[col_i.at[c]], buf.at[:, pl.ds(HL + HR, HR)], sem)
        return c1, c2, c3

    def wait_gather(c, buf, sem):
        pltpu.make_async_copy(
            lab_w_hbm.at[lab_i.at[c]], buf.at[:, pl.ds(0, HL)], sem).wait()
        pltpu.make_async_copy(
            row_w_hbm.at[row_i.at[c]], buf.at[:, pl.ds(HL, HR)], sem).wait()
        pltpu.make_async_copy(
            col_w_hbm.at[col_i.at[c]], buf.at[:, pl.ds(HL + HR, HR)],
            sem).wait()

    gather(0, buf0, gsem0)
    gather(1, buf1, gsem1)

    def body(g, carry):
        c0 = 2 * g
        c1 = c0 + 1
        wait_gather(c0, buf0, gsem0)
        s0 = pltpu.async_copy(
            buf0, out_hbm.at[pl.ds(wid * BW + c0 * R, R)], ssem0)

        wait_gather(c1, buf1, gsem1)
        s1 = pltpu.async_copy(
            buf1, out_hbm.at[pl.ds(wid * BW + c1 * R, R)], ssem1)

        s0.wait()

        @pl.when(c0 + 2 < NCH)
        def _():
            gather(c0 + 2, buf0, gsem0)

        s1.wait()

        @pl.when(c1 + 2 < NCH)
        def _():
            gather(c1 + 2, buf1, gsem1)

        return carry

    lax.fori_loop(0, NCH // 2, body, 0)


def kernel(label, label_logits, row_id, column_id, epoch, label_emb_w,
           row_emb_w, col_emb_w):
    del label_logits, epoch
    lab_i = label.astype(jnp.int32).reshape(NW, NCH, R)
    row_i = row_id.astype(jnp.int32).reshape(NW, NCH, R)
    col_i = column_id.astype(jnp.int32).reshape(NW, NCH, R)
    return _sc_embed(lab_i, row_i, col_i, label_emb_w, row_emb_w, col_emb_w)


# speedup vs baseline: 1.4242x; 1.4242x over previous
import functools

import jax
import jax.numpy as jnp
from jax import lax
from jax.experimental import pallas as pl
from jax.experimental.pallas import tpu as pltpu
from jax.experimental.pallas import tpu_sc as plsc

S = 16384
HL = 1024
HR = 256
W = HL + 2 * HR
L = 16

NC = 2
NS = 16
NW = NC * NS
BW = S // NW
RPC = 16
NCH = BW // RPC


@functools.partial(
    pl.kernel,
    mesh=plsc.VectorSubcoreMesh(core_axis_name="c", subcore_axis_name="s"),
    compiler_params=pltpu.CompilerParams(needs_layout_passes=False),
    out_type=jax.ShapeDtypeStruct((S * W,), jnp.float32),
    scratch_types=[
        pltpu.VMEM((BW,), jnp.int32),
        pltpu.VMEM((BW,), jnp.int32),
        pltpu.VMEM((BW,), jnp.int32),
        pltpu.VMEM((5 * HL,), jnp.float32),
        pltpu.VMEM((50 * HR,), jnp.float32),
        pltpu.VMEM((50 * HR,), jnp.float32),
        pltpu.VMEM((RPC * W,), jnp.float32),
        pltpu.VMEM((RPC * W,), jnp.float32),
        pltpu.SemaphoreType.DMA,
        pltpu.SemaphoreType.DMA,
    ],
)
def _sc_embed(lab_i_hbm, row_i_hbm, col_i_hbm, lab_w_hbm, row_w_hbm,
              col_w_hbm, out_hbm, lab_i, row_i, col_i, lab_w_v, row_w_v,
              col_w_v, buf0, buf1, ssem0, ssem1):
    wid = lax.axis_index("s") * NC + lax.axis_index("c")
    pltpu.sync_copy(lab_i_hbm.at[wid], lab_i)
    pltpu.sync_copy(row_i_hbm.at[wid], row_i)
    pltpu.sync_copy(col_i_hbm.at[wid], col_i)
    pltpu.sync_copy(lab_w_hbm, lab_w_v)
    pltpu.sync_copy(row_w_hbm, row_w_v)
    pltpu.sync_copy(col_w_hbm, col_w_v)

    iota = lax.iota(jnp.int32, L)
    dnums = lax.GatherDimensionNumbers(
        offset_dims=(), collapsed_slice_dims=(0,), start_index_map=(0,))

    def splat(vec, rv):
        return lax.gather(
            vec, rv[:, None], dnums, (1,),
            mode=lax.GatherScatterMode.PROMISE_IN_BOUNDS)

    def assemble(c, buf):
        lab16 = lab_i[pl.ds(c * RPC, L)]
        row16 = row_i[pl.ds(c * RPC, L)]
        col16 = col_i[pl.ds(c * RPC, L)]

        def rowbody(r, carry):
            rv = jnp.full((L,), 0, jnp.int32) + r
            lab_a = splat(lab16, rv) * HL + iota
            row_a = splat(row16, rv) * HR + iota
            col_a = splat(col16, rv) * HR + iota
            rbase = r * W
            for k in range(HL // L):
                v = plsc.load_gather(lab_w_v, [lab_a + L * k])
                buf[pl.ds(rbase + L * k, L)] = v
            for k in range(HR // L):
                v = plsc.load_gather(row_w_v, [row_a + L * k])
                buf[pl.ds(rbase + HL + L * k, L)] = v
                v = plsc.load_gather(col_w_v, [col_a + L * k])
                buf[pl.ds(rbase + HL + HR + L * k, L)] = v
            return carry

        lax.fori_loop(0, RPC, rowbody, 0)

    def out_slab(c):
        return out_hbm.at[pl.ds((wid * BW + c * RPC) * W, RPC * W)]

    def body(g, carry):
        c0 = 2 * g
        c1 = c0 + 1

        @pl.when(g > 0)
        def _():
            pltpu.make_async_copy(buf0, out_slab(c0 - 2), ssem0).wait()

        assemble(c0, buf0)
        pltpu.async_copy(buf0, out_slab(c0), ssem0)

        @pl.when(g > 0)
        def _():
            pltpu.make_async_copy(buf1, out_slab(c1 - 2), ssem1).wait()

        assemble(c1, buf1)
        pltpu.async_copy(buf1, out_slab(c1), ssem1)
        return carry

    lax.fori_loop(0, NCH // 2, body, 0)
    pltpu.make_async_copy(buf0, out_slab(NCH - 2), ssem0).wait()
    pltpu.make_async_copy(buf1, out_slab(NCH - 1), ssem1).wait()


def kernel(label, label_logits, row_id, column_id, epoch, label_emb_w,
           row_emb_w, col_emb_w):
    del label_logits, epoch
    lab_i = label.astype(jnp.int32).reshape(NW, BW)
    row_i = row_id.astype(jnp.int32).reshape(NW, BW)
    col_i = column_id.astype(jnp.int32).reshape(NW, BW)
    out = _sc_embed(lab_i, row_i, col_i, label_emb_w.reshape(-1),
                    row_emb_w.reshape(-1), col_emb_w.reshape(-1))
    return out.reshape(S, W)


# speedup vs baseline: 1.5025x; 1.0550x over previous
import functools

import jax
import jax.numpy as jnp
from jax import lax
from jax.experimental import pallas as pl
from jax.experimental.pallas import tpu as pltpu
from jax.experimental.pallas import tpu_sc as plsc

S = 16384
HL = 1024
HR = 256
W = HL + 2 * HR
L = 16

NC = 2
NS = 16
NW = NC * NS
BW = S // NW
RPC = 16
NCH = BW // RPC


@functools.partial(
    pl.kernel,
    mesh=plsc.VectorSubcoreMesh(core_axis_name="c", subcore_axis_name="s"),
    compiler_params=pltpu.CompilerParams(needs_layout_passes=False),
    out_type=jax.ShapeDtypeStruct((S * W,), jnp.float32),
    scratch_types=[
        pltpu.SMEM((BW,), jnp.int32),
        pltpu.SMEM((BW,), jnp.int32),
        pltpu.SMEM((BW,), jnp.int32),
        pltpu.VMEM((BW,), jnp.int32),
        pltpu.VMEM((5 * HL,), jnp.float32),
        pltpu.VMEM((50 * HR,), jnp.float32),
        pltpu.VMEM((50 * HR,), jnp.float32),
        pltpu.VMEM((RPC * W,), jnp.float32),
        pltpu.VMEM((RPC * W,), jnp.float32),
        pltpu.SemaphoreType.DMA,
        pltpu.SemaphoreType.DMA,
    ],
)
def _sc_embed(lab_i_hbm, row_i_hbm, col_i_hbm, lab_w_hbm, row_w_hbm,
              col_w_hbm, out_hbm, lab_i, row_i, col_i, idx_v, lab_w_v,
              row_w_v, col_w_v, buf0, buf1, ssem0, ssem1):
    wid = lax.axis_index("s") * NC + lax.axis_index("c")
    iota = lax.iota(jnp.int32, L)

    def to_smem(src_hbm, sm):
        pltpu.sync_copy(src_hbm, idx_v)

        def vbody(vc, carry):
            v = idx_v[pl.ds(vc * L, L)]
            for r in range(L):
                sm[vc * L + r] = jnp.sum(jnp.where(iota == r, v, 0))
            return carry

        lax.fori_loop(0, BW // L, vbody, 0)

    to_smem(lab_i_hbm.at[wid], lab_i)
    to_smem(row_i_hbm.at[wid], row_i)
    to_smem(col_i_hbm.at[wid], col_i)
    pltpu.sync_copy(lab_w_hbm, lab_w_v)
    pltpu.sync_copy(row_w_hbm, row_w_v)
    pltpu.sync_copy(col_w_hbm, col_w_v)

    def assemble(c, buf):
        def rowbody(r, carry):
            i = c * RPC + r
            ls = lab_i[i] * HL
            rs = row_i[i] * HR
            cs = col_i[i] * HR
            rbase = r * W
            for k in range(HL // L):
                buf[pl.ds(rbase + L * k, L)] = lab_w_v[pl.ds(ls + L * k, L)]
            for k in range(HR // L):
                buf[pl.ds(rbase + HL + L * k, L)] = (
                    row_w_v[pl.ds(rs + L * k, L)])
                buf[pl.ds(rbase + HL + HR + L * k, L)] = (
                    col_w_v[pl.ds(cs + L * k, L)])
            return carry

        lax.fori_loop(0, RPC, rowbody, 0)

    def out_slab(c):
        return out_hbm.at[pl.ds((wid * BW + c * RPC) * W, RPC * W)]

    def body(g, carry):
        c0 = 2 * g
        c1 = c0 + 1

        @pl.when(g > 0)
        def _():
            pltpu.make_async_copy(buf0, out_slab(c0 - 2), ssem0).wait()

        assemble(c0, buf0)
        pltpu.async_copy(buf0, out_slab(c0), ssem0)

        @pl.when(g > 0)
        def _():
            pltpu.make_async_copy(buf1, out_slab(c1 - 2), ssem1).wait()

        assemble(c1, buf1)
        pltpu.async_copy(buf1, out_slab(c1), ssem1)
        return carry

    lax.fori_loop(0, NCH // 2, body, 0)
    pltpu.make_async_copy(buf0, out_slab(NCH - 2), ssem0).wait()
    pltpu.make_async_copy(buf1, out_slab(NCH - 1), ssem1).wait()


def kernel(label, label_logits, row_id, column_id, epoch, label_emb_w,
           row_emb_w, col_emb_w):
    del label_logits, epoch
    lab_i = label.astype(jnp.int32).reshape(NW, BW)
    row_i = row_id.astype(jnp.int32).reshape(NW, BW)
    col_i = column_id.astype(jnp.int32).reshape(NW, BW)
    out = _sc_embed(lab_i, row_i, col_i, label_emb_w.reshape(-1),
                    row_emb_w.reshape(-1), col_emb_w.reshape(-1))
    return out.reshape(S, W)


# speedup vs baseline: 2.3882x; 1.5894x over previous
import functools

import jax
import jax.numpy as jnp
from jax import lax
from jax.experimental import pallas as pl
from jax.experimental.pallas import tpu as pltpu
from jax.experimental.pallas import tpu_sc as plsc

S = 16384
HL = 1024
HR = 256
W = HL + 2 * HR
L = 16

NC = 2
NS = 16
NW = NC * NS
BW = S // NW
RPC = 16
NCH = BW // RPC


@functools.partial(
    pl.kernel,
    mesh=plsc.VectorSubcoreMesh(core_axis_name="c", subcore_axis_name="s"),
    compiler_params=pltpu.CompilerParams(needs_layout_passes=False),
    out_type=jax.ShapeDtypeStruct((S * W,), jnp.float32),
    scratch_types=[
        pltpu.SMEM((BW,), jnp.int32),
        pltpu.SMEM((BW,), jnp.int32),
        pltpu.SMEM((BW,), jnp.int32),
        pltpu.VMEM((BW,), jnp.int32),
        pltpu.VMEM((5 * HL,), jnp.float32),
        pltpu.VMEM((50 * HR,), jnp.float32),
        pltpu.VMEM((50 * HR,), jnp.float32),
        pltpu.VMEM((RPC * W,), jnp.float32),
        pltpu.VMEM((RPC * W,), jnp.float32),
        pltpu.SemaphoreType.DMA,
        pltpu.SemaphoreType.DMA,
    ],
)
def _sc_embed(lab_i_hbm, row_i_hbm, col_i_hbm, lab_w_hbm, row_w_hbm,
              col_w_hbm, out_hbm, lab_i, row_i, col_i, idx_v, lab_w_v,
              row_w_v, col_w_v, buf0, buf1, ssem0, ssem1):
    wid = lax.axis_index("s") * NC + lax.axis_index("c")
    iota = lax.iota(jnp.int32, L)

    def to_smem(src_hbm, sm):
        pltpu.sync_copy(src_hbm, idx_v)

        def vbody(vc, carry):
            v = idx_v[pl.ds(vc * L, L)]
            for r in range(L):
                sm[vc * L + r] = jnp.sum(jnp.where(iota == r, v, 0))
            return carry

        lax.fori_loop(0, BW // L, vbody, 0)

    to_smem(lab_i_hbm.at[wid], lab_i)
    to_smem(row_i_hbm.at[wid], row_i)
    to_smem(col_i_hbm.at[wid], col_i)
    pltpu.sync_copy(lab_w_hbm, lab_w_v)
    pltpu.sync_copy(row_w_hbm, row_w_v)
    pltpu.sync_copy(col_w_hbm, col_w_v)

    def assemble(c, buf):
        @plsc.parallel_loop(0, RPC, unroll=2)
        def rowbody(r):
            i = c * RPC + r
            ls = lab_i[i] * HL
            rs = row_i[i] * HR
            cs = col_i[i] * HR
            rbase = r * W
            for k in range(HL // L):
                buf[pl.ds(rbase + L * k, L)] = lab_w_v[pl.ds(ls + L * k, L)]
            for k in range(HR // L):
                buf[pl.ds(rbase + HL + L * k, L)] = (
                    row_w_v[pl.ds(rs + L * k, L)])
                buf[pl.ds(rbase + HL + HR + L * k, L)] = (
                    col_w_v[pl.ds(cs + L * k, L)])

    def out_slab(c):
        return out_hbm.at[pl.ds((wid * BW + c * RPC) * W, RPC * W)]

    def body(g, carry):
        c0 = 2 * g
        c1 = c0 + 1

        @pl.when(g > 0)
        def _():
            pltpu.make_async_copy(buf0, out_slab(c0 - 2), ssem0).wait()

        assemble(c0, buf0)
        pltpu.async_copy(buf0, out_slab(c0), ssem0)

        @pl.when(g > 0)
        def _():
            pltpu.make_async_copy(buf1, out_slab(c1 - 2), ssem1).wait()

        assemble(c1, buf1)
        pltpu.async_copy(buf1, out_slab(c1), ssem1)
        return carry

    lax.fori_loop(0, NCH // 2, body, 0)
    pltpu.make_async_copy(buf0, out_slab(NCH - 2), ssem0).wait()
    pltpu.make_async_copy(buf1, out_slab(NCH - 1), ssem1).wait()


def kernel(label, label_logits, row_id, column_id, epoch, label_emb_w,
           row_emb_w, col_emb_w):
    del label_logits, epoch
    lab_i = label.astype(jnp.int32).reshape(NW, BW)
    row_i = row_id.astype(jnp.int32).reshape(NW, BW)
    col_i = column_id.astype(jnp.int32).reshape(NW, BW)
    out = _sc_embed(lab_i, row_i, col_i, label_emb_w.reshape(-1),
                    row_emb_w.reshape(-1), col_emb_w.reshape(-1))
    return out.reshape(S, W)
